# baseline (speedup 1.0000x reference)
import functools

import jax
import jax.numpy as jnp
from jax import lax
from jax.experimental import pallas as pl
from jax.experimental.pallas import tpu as pltpu
from jax.experimental.pallas import tpu_sc as plsc

NUM_EXPERTS = 8
TOP_K = 2
BLK = 256
NBLK = 24
NPAD = NBLK * BLK
CHUNK = 128


def _router_meta_body(x_ref, wr_ref, pos0_ref, pos1_ref, w0_ref, w1_ref,
                      meta_ref):
    x = x_ref[...]
    logits = jnp.dot(x, wr_ref[...], preferred_element_type=jnp.float32)
    t, ne = logits.shape
    eids = lax.broadcasted_iota(jnp.int32, (t, ne), 1)
    m1 = jnp.max(logits, axis=1, keepdims=True)
    a1 = jnp.min(jnp.where(logits == m1, eids, ne), axis=1, keepdims=True)
    h1 = eids == a1
    masked = jnp.where(h1, -jnp.inf, logits)
    m2 = jnp.max(masked, axis=1, keepdims=True)
    a2 = jnp.min(jnp.where(masked == m2, eids, ne), axis=1, keepdims=True)
    h2 = eids == a2
    e2 = jnp.exp(m2 - m1)
    w0_ref[...] = 1.0 / (1.0 + e2)
    w1_ref[...] = e2 / (1.0 + e2)

    cnt = jnp.where(h1, 1.0, 0.0) + jnp.where(h2, 1.0, 0.0)
    r = lax.broadcasted_iota(jnp.int32, (CHUNK, CHUNK), 0)
    c = lax.broadcasted_iota(jnp.int32, (CHUNK, CHUNK), 1)
    ltri = jnp.where(r > c, 1.0, 0.0).astype(jnp.bfloat16)
    nch = t // CHUNK
    incs = []
    tots = []
    for ci in range(nch):
        sl = cnt[ci * CHUNK:(ci + 1) * CHUNK, :]
        incs.append(jnp.dot(ltri, sl.astype(jnp.bfloat16),
                            preferred_element_type=jnp.float32))
        tots.append(jnp.sum(sl, axis=0, keepdims=True))
    tots = jnp.concatenate(tots, axis=0)
    r16 = lax.broadcasted_iota(jnp.int32, (nch, nch), 0)
    c16 = lax.broadcasted_iota(jnp.int32, (nch, nch), 1)
    ltri16 = jnp.where(r16 > c16, 1.0, 0.0).astype(jnp.bfloat16)
    base = jnp.dot(ltri16, tots.astype(jnp.bfloat16),
                   preferred_element_type=jnp.float32)
    cum = jnp.concatenate(
        [incs[ci] + base[ci:ci + 1, :] for ci in range(nch)], axis=0)

    counts = jnp.sum(tots, axis=0, keepdims=True)
    nblk = jnp.floor((counts + (BLK - 1)) / BLK)
    rc = lax.broadcasted_iota(jnp.int32, (NUM_EXPERTS, NUM_EXPERTS), 0)
    cc = lax.broadcasted_iota(jnp.int32, (NUM_EXPERTS, NUM_EXPERTS), 1)
    utri = jnp.where(rc < cc, 1.0, 0.0).astype(jnp.bfloat16)
    bstart = jnp.dot(nblk.astype(jnp.bfloat16), utri,
                     preferred_element_type=jnp.float32)
    po = bstart * BLK

    dest = po + cum
    pos0_ref[...] = jnp.sum(jnp.where(h1, dest, 0.0), axis=1,
                            keepdims=True).astype(jnp.int32)
    pos1_ref[...] = jnp.sum(jnp.where(h2, dest, 0.0), axis=1,
                            keepdims=True).astype(jnp.int32)

    nb_used = jnp.sum(nblk, axis=1, keepdims=True)
    bi = lax.broadcasted_iota(jnp.int32, (32, NUM_EXPERTS), 0).astype(
        jnp.float32)
    be = jnp.sum(jnp.where(bi >= bstart, 1.0, 0.0), axis=1,
                 keepdims=True) - 1.0
    bcol = lax.broadcasted_iota(jnp.int32, (32, 1), 0)
    meta_ref[...] = jnp.where(bcol < NBLK, be, nb_used).astype(jnp.int32)


def _gmm_body(meta_ref, xs_ref, w_ref, b_ref, y_ref):
    b = pl.program_id(0)

    @pl.when(b < meta_ref[NBLK])
    def _():
        y_ref[...] = (jnp.dot(xs_ref[...], w_ref[0],
                              preferred_element_type=jnp.float32)
                      + b_ref[0]).astype(jnp.bfloat16)


def _fma_body(g0_ref, g1_ref, w0_ref, w1_ref, out_ref):
    out_ref[...] = (g0_ref[...].astype(jnp.float32) * w0_ref[...]
                    + g1_ref[...].astype(jnp.float32) * w1_ref[...])


@jax.jit
def kernel(X, W_router, W_experts, b_experts):
    B, T, D = X.shape
    x2 = X.reshape(T, D).astype(jnp.bfloat16)
    we_bf = W_experts.astype(jnp.bfloat16)
    wr_bf = W_router.astype(jnp.bfloat16)

    pos0, pos1, w0, w1, meta = pl.pallas_call(
        _router_meta_body,
        in_specs=[
            pl.BlockSpec((T, D), lambda: (0, 0)),
            pl.BlockSpec((D, NUM_EXPERTS), lambda: (0, 0)),
        ],
        out_specs=[
            pl.BlockSpec((T, 1), lambda: (0, 0)),
            pl.BlockSpec((T, 1), lambda: (0, 0)),
            pl.BlockSpec((T, 1), lambda: (0, 0)),
            pl.BlockSpec((T, 1), lambda: (0, 0)),
            pl.BlockSpec((32, 1), lambda: (0, 0)),
        ],
        out_shape=[
            jax.ShapeDtypeStruct((T, 1), jnp.int32),
            jax.ShapeDtypeStruct((T, 1), jnp.int32),
            jax.ShapeDtypeStruct((T, 1), jnp.float32),
            jax.ShapeDtypeStruct((T, 1), jnp.float32),
            jax.ShapeDtypeStruct((32, 1), jnp.int32),
        ],
    )(x2, wr_bf)

    info = plsc.get_sparse_core_info()
    nc, ns = info.num_cores, info.num_subcores
    nw = nc * ns
    tpw = T // nw
    d2 = D // 2
    mesh = plsc.VectorSubcoreMesh(core_axis_name="c", subcore_axis_name="s")
    pos0w = pos0.reshape(nw, tpw)
    pos1w = pos1.reshape(nw, tpw)
    xf = lax.bitcast_convert_type(x2.reshape(T, d2, 2), jnp.float32)

    @functools.partial(
        pl.kernel, mesh=mesh,
        out_type=jax.ShapeDtypeStruct((NPAD, d2), jnp.float32),
        scratch_types=[
            pltpu.VMEM((tpw,), jnp.int32),
            pltpu.VMEM((tpw, d2), jnp.float32),
            pltpu.SemaphoreType.DMA,
        ],
    )
    def dispatch(x_hbm, p0_hbm, p1_hbm, xs_hbm, idx_v, rows_v, sem):
        wid = lax.axis_index("s") * nc + lax.axis_index("c")
        base = wid * tpw
        pltpu.sync_copy(x_hbm.at[pl.ds(base, tpw)], rows_v)
        pltpu.sync_copy(p0_hbm.at[wid], idx_v)
        pltpu.async_copy(rows_v, xs_hbm.at[idx_v], sem).wait()
        pltpu.sync_copy(p1_hbm.at[wid], idx_v)
        pltpu.async_copy(rows_v, xs_hbm.at[idx_v], sem).wait()

    xs = dispatch(xf, pos0w, pos1w)
    xs = lax.bitcast_convert_type(xs, jnp.bfloat16).reshape(NPAD, D)

    y = pl.pallas_call(
        _gmm_body,
        grid_spec=pltpu.PrefetchScalarGridSpec(
            num_scalar_prefetch=1,
            grid=(NBLK,),
            in_specs=[
                pl.BlockSpec((BLK, D), lambda b, m: (b, 0)),
                pl.BlockSpec((1, D, D), lambda b, m: (m[b], 0, 0)),
                pl.BlockSpec((1, 1, D), lambda b, m: (m[b], 0, 0)),
            ],
            out_specs=pl.BlockSpec((BLK, D), lambda b, m: (b, 0)),
        ),
        out_shape=jax.ShapeDtypeStruct((NPAD, D), jnp.bfloat16),
    )(meta.reshape(32), xs, we_bf, b_experts.reshape(NUM_EXPERTS, 1, D))

    @functools.partial(
        pl.kernel, mesh=mesh,
        out_type=(jax.ShapeDtypeStruct((T, d2), jnp.float32),
                  jax.ShapeDtypeStruct((T, d2), jnp.float32)),
        scratch_types=[
            pltpu.VMEM((tpw,), jnp.int32),
            pltpu.VMEM((tpw, d2), jnp.float32),
            pltpu.SemaphoreType.DMA,
        ],
    )
    def combine_gather(y_hbm, p0_hbm, p1_hbm, g0_hbm, g1_hbm,
                       idx_v, rows_v, sem):
        wid = lax.axis_index("s") * nc + lax.axis_index("c")
        base = wid * tpw
        pltpu.sync_copy(p0_hbm.at[wid], idx_v)
        pltpu.async_copy(y_hbm.at[idx_v], rows_v, sem).wait()
        pltpu.sync_copy(rows_v, g0_hbm.at[pl.ds(base, tpw)])
        pltpu.sync_copy(p1_hbm.at[wid], idx_v)
        pltpu.async_copy(y_hbm.at[idx_v], rows_v, sem).wait()
        pltpu.sync_copy(rows_v, g1_hbm.at[pl.ds(base, tpw)])

    yf = lax.bitcast_convert_type(y.reshape(NPAD, d2, 2), jnp.float32)
    g0, g1 = combine_gather(yf, pos0w, pos1w)
    g0 = lax.bitcast_convert_type(g0, jnp.bfloat16).reshape(T, D)
    g1 = lax.bitcast_convert_type(g1, jnp.bfloat16).reshape(T, D)

    out = pl.pallas_call(
        _fma_body,
        in_specs=[
            pl.BlockSpec((T, D), lambda: (0, 0)),
            pl.BlockSpec((T, D), lambda: (0, 0)),
            pl.BlockSpec((T, 1), lambda: (0, 0)),
            pl.BlockSpec((T, 1), lambda: (0, 0)),
        ],
        out_specs=pl.BlockSpec((T, D), lambda: (0, 0)),
        out_shape=jax.ShapeDtypeStruct((T, D), jnp.float32),
    )(g0, g1, w0, w1)

    return out.reshape(B, T, D)


# speedup vs baseline: 4.8784x; 4.8784x over previous
import functools

import jax
import jax.numpy as jnp
from jax import lax
from jax.experimental import pallas as pl
from jax.experimental.pallas import tpu as pltpu
from jax.experimental.pallas import tpu_sc as plsc

NUM_EXPERTS = 8
TOP_K = 2
BLK = 256
NBLK = 24
NPAD = NBLK * BLK
CHUNK = 128


def _router_meta_body(x_ref, wr_ref, pos0_ref, pos1_ref, w0_ref, w1_ref,
                      meta_ref):
    x = x_ref[...]
    logits = jnp.dot(x, wr_ref[...], preferred_element_type=jnp.float32)
    t, ne = logits.shape
    eids = lax.broadcasted_iota(jnp.int32, (t, ne), 1)
    m1 = jnp.max(logits, axis=1, keepdims=True)
    a1 = jnp.min(jnp.where(logits == m1, eids, ne), axis=1, keepdims=True)
    h1 = eids == a1
    masked = jnp.where(h1, -jnp.inf, logits)
    m2 = jnp.max(masked, axis=1, keepdims=True)
    a2 = jnp.min(jnp.where(masked == m2, eids, ne), axis=1, keepdims=True)
    h2 = eids == a2
    e2 = jnp.exp(m2 - m1)
    w0_ref[...] = 1.0 / (1.0 + e2)
    w1_ref[...] = e2 / (1.0 + e2)

    cnt = jnp.where(h1, 1.0, 0.0) + jnp.where(h2, 1.0, 0.0)
    r = lax.broadcasted_iota(jnp.int32, (CHUNK, CHUNK), 0)
    c = lax.broadcasted_iota(jnp.int32, (CHUNK, CHUNK), 1)
    ltri = jnp.where(r > c, 1.0, 0.0).astype(jnp.bfloat16)
    nch = t // CHUNK
    incs = []
    tots = []
    for ci in range(nch):
        sl = cnt[ci * CHUNK:(ci + 1) * CHUNK, :]
        incs.append(jnp.dot(ltri, sl.astype(jnp.bfloat16),
                            preferred_element_type=jnp.float32))
        tots.append(jnp.sum(sl, axis=0, keepdims=True))
    tots = jnp.concatenate(tots, axis=0)
    r16 = lax.broadcasted_iota(jnp.int32, (nch, nch), 0)
    c16 = lax.broadcasted_iota(jnp.int32, (nch, nch), 1)
    ltri16 = jnp.where(r16 > c16, 1.0, 0.0).astype(jnp.bfloat16)
    base = jnp.dot(ltri16, tots.astype(jnp.bfloat16),
                   preferred_element_type=jnp.float32)
    cum = jnp.concatenate(
        [incs[ci] + base[ci:ci + 1, :] for ci in range(nch)], axis=0)

    counts = jnp.sum(tots, axis=0, keepdims=True)
    nblk = jnp.floor((counts + (BLK - 1)) / BLK)
    rc = lax.broadcasted_iota(jnp.int32, (NUM_EXPERTS, NUM_EXPERTS), 0)
    cc = lax.broadcasted_iota(jnp.int32, (NUM_EXPERTS, NUM_EXPERTS), 1)
    utri = jnp.where(rc < cc, 1.0, 0.0).astype(jnp.bfloat16)
    bstart = jnp.dot(nblk.astype(jnp.bfloat16), utri,
                     preferred_element_type=jnp.float32)
    po = bstart * BLK

    dest = po + cum
    pos0_ref[...] = jnp.sum(jnp.where(h1, dest, 0.0), axis=1,
                            keepdims=True).astype(jnp.int32)
    pos1_ref[...] = jnp.sum(jnp.where(h2, dest, 0.0), axis=1,
                            keepdims=True).astype(jnp.int32)

    nb_used = jnp.sum(nblk, axis=1, keepdims=True)
    bi = lax.broadcasted_iota(jnp.int32, (32, NUM_EXPERTS), 0).astype(
        jnp.float32)
    be = jnp.sum(jnp.where(bi >= bstart, 1.0, 0.0), axis=1,
                 keepdims=True) - 1.0
    bcol = lax.broadcasted_iota(jnp.int32, (32, 1), 0)
    meta_ref[...] = jnp.where(bcol < NBLK, be, nb_used).astype(jnp.int32)


def _gmm_body(meta_ref, xs_ref, w_ref, b_ref, y_ref):
    b = pl.program_id(0)

    @pl.when(b < meta_ref[NBLK])
    def _():
        y_ref[...] = jnp.dot(xs_ref[...], w_ref[0],
                             precision=lax.Precision.DEFAULT,
                             preferred_element_type=jnp.float32) + b_ref[0]


def _fma_body(g0_ref, g1_ref, w0_ref, w1_ref, out_ref):
    out_ref[...] = g0_ref[...] * w0_ref[...] + g1_ref[...] * w1_ref[...]


@jax.jit
def kernel(X, W_router, W_experts, b_experts):
    B, T, D = X.shape
    x2 = X.reshape(T, D)

    pos0, pos1, w0, w1, meta = pl.pallas_call(
        _router_meta_body,
        in_specs=[
            pl.BlockSpec((T, D), lambda: (0, 0)),
            pl.BlockSpec((D, NUM_EXPERTS), lambda: (0, 0)),
        ],
        out_specs=[
            pl.BlockSpec((T, 1), lambda: (0, 0)),
            pl.BlockSpec((T, 1), lambda: (0, 0)),
            pl.BlockSpec((T, 1), lambda: (0, 0)),
            pl.BlockSpec((T, 1), lambda: (0, 0)),
            pl.BlockSpec((32, 1), lambda: (0, 0)),
        ],
        out_shape=[
            jax.ShapeDtypeStruct((T, 1), jnp.int32),
            jax.ShapeDtypeStruct((T, 1), jnp.int32),
            jax.ShapeDtypeStruct((T, 1), jnp.float32),
            jax.ShapeDtypeStruct((T, 1), jnp.float32),
            jax.ShapeDtypeStruct((32, 1), jnp.int32),
        ],
    )(x2, W_router)

    info = plsc.get_sparse_core_info()
    nc, ns = info.num_cores, info.num_subcores
    nw = nc * ns
    tpw = T // nw
    mesh = plsc.VectorSubcoreMesh(core_axis_name="c", subcore_axis_name="s")
    pos0w = pos0.reshape(nw, tpw)
    pos1w = pos1.reshape(nw, tpw)

    @functools.partial(
        pl.kernel, mesh=mesh,
        out_type=jax.ShapeDtypeStruct((NPAD, D), jnp.float32),
        scratch_types=[
            pltpu.VMEM((tpw,), jnp.int32),
            pltpu.VMEM((tpw, D), jnp.float32),
            pltpu.SemaphoreType.DMA,
        ],
    )
    def dispatch(x_hbm, p0_hbm, p1_hbm, xs_hbm, idx_v, rows_v, sem):
        wid = lax.axis_index("s") * nc + lax.axis_index("c")
        base = wid * tpw
        pltpu.sync_copy(x_hbm.at[pl.ds(base, tpw)], rows_v)
        pltpu.sync_copy(p0_hbm.at[wid], idx_v)
        pltpu.async_copy(rows_v, xs_hbm.at[idx_v], sem).wait()
        pltpu.sync_copy(p1_hbm.at[wid], idx_v)
        pltpu.async_copy(rows_v, xs_hbm.at[idx_v], sem).wait()

    xs = dispatch(x2, pos0w, pos1w)

    y = pl.pallas_call(
        _gmm_body,
        grid_spec=pltpu.PrefetchScalarGridSpec(
            num_scalar_prefetch=1,
            grid=(NBLK,),
            in_specs=[
                pl.BlockSpec((BLK, D), lambda b, m: (b, 0)),
                pl.BlockSpec((1, D, D), lambda b, m: (m[b], 0, 0)),
                pl.BlockSpec((1, 1, D), lambda b, m: (m[b], 0, 0)),
            ],
            out_specs=pl.BlockSpec((BLK, D), lambda b, m: (b, 0)),
        ),
        out_shape=jax.ShapeDtypeStruct((NPAD, D), jnp.float32),
    )(meta.reshape(32), xs, W_experts, b_experts.reshape(NUM_EXPERTS, 1, D))

    @functools.partial(
        pl.kernel, mesh=mesh,
        out_type=(jax.ShapeDtypeStruct((T, D), jnp.float32),
                  jax.ShapeDtypeStruct((T, D), jnp.float32)),
        scratch_types=[
            pltpu.VMEM((tpw,), jnp.int32),
            pltpu.VMEM((tpw, D), jnp.float32),
            pltpu.SemaphoreType.DMA,
        ],
    )
    def combine_gather(y_hbm, p0_hbm, p1_hbm, g0_hbm, g1_hbm,
                       idx_v, rows_v, sem):
        wid = lax.axis_index("s") * nc + lax.axis_index("c")
        base = wid * tpw
        pltpu.sync_copy(p0_hbm.at[wid], idx_v)
        pltpu.async_copy(y_hbm.at[idx_v], rows_v, sem).wait()
        pltpu.sync_copy(rows_v, g0_hbm.at[pl.ds(base, tpw)])
        pltpu.sync_copy(p1_hbm.at[wid], idx_v)
        pltpu.async_copy(y_hbm.at[idx_v], rows_v, sem).wait()
        pltpu.sync_copy(rows_v, g1_hbm.at[pl.ds(base, tpw)])

    g0, g1 = combine_gather(y, pos0w, pos1w)

    out = pl.pallas_call(
        _fma_body,
        in_specs=[
            pl.BlockSpec((T, D), lambda: (0, 0)),
            pl.BlockSpec((T, D), lambda: (0, 0)),
            pl.BlockSpec((T, 1), lambda: (0, 0)),
            pl.BlockSpec((T, 1), lambda: (0, 0)),
        ],
        out_specs=pl.BlockSpec((T, D), lambda: (0, 0)),
        out_shape=jax.ShapeDtypeStruct((T, D), jnp.float32),
    )(g0, g1, w0, w1)

    return out.reshape(B, T, D)


# speedup vs baseline: 4.9459x; 1.0138x over previous
import functools

import jax
import jax.numpy as jnp
from jax import lax
from jax.experimental import pallas as pl
from jax.experimental.pallas import tpu as pltpu
from jax.experimental.pallas import tpu_sc as plsc

NUM_EXPERTS = 8
TOP_K = 2
BLK = 256
NBLK = 24
NPAD = NBLK * BLK
CHUNK = 128


def _router_meta_body(x_ref, wr_ref, pos0_ref, pos1_ref, w0_ref, w1_ref,
                      meta_ref):
    x = x_ref[...]
    logits = jnp.dot(x, wr_ref[...], preferred_element_type=jnp.float32)
    t, ne = logits.shape
    eids = lax.broadcasted_iota(jnp.int32, (t, ne), 1)
    m1 = jnp.max(logits, axis=1, keepdims=True)
    a1 = jnp.min(jnp.where(logits == m1, eids, ne), axis=1, keepdims=True)
    h1 = eids == a1
    masked = jnp.where(h1, -jnp.inf, logits)
    m2 = jnp.max(masked, axis=1, keepdims=True)
    a2 = jnp.min(jnp.where(masked == m2, eids, ne), axis=1, keepdims=True)
    h2 = eids == a2
    e2 = jnp.exp(m2 - m1)
    w0_ref[...] = 1.0 / (1.0 + e2)
    w1_ref[...] = e2 / (1.0 + e2)

    cnt = jnp.where(h1, 1.0, 0.0) + jnp.where(h2, 1.0, 0.0)
    r = lax.broadcasted_iota(jnp.int32, (CHUNK, CHUNK), 0)
    c = lax.broadcasted_iota(jnp.int32, (CHUNK, CHUNK), 1)
    ltri = jnp.where(r > c, 1.0, 0.0).astype(jnp.bfloat16)
    nch = t // CHUNK
    incs = []
    tots = []
    for ci in range(nch):
        sl = cnt[ci * CHUNK:(ci + 1) * CHUNK, :]
        incs.append(jnp.dot(ltri, sl.astype(jnp.bfloat16),
                            preferred_element_type=jnp.float32))
        tots.append(jnp.sum(sl, axis=0, keepdims=True))
    tots = jnp.concatenate(tots, axis=0)
    r16 = lax.broadcasted_iota(jnp.int32, (nch, nch), 0)
    c16 = lax.broadcasted_iota(jnp.int32, (nch, nch), 1)
    ltri16 = jnp.where(r16 > c16, 1.0, 0.0).astype(jnp.bfloat16)
    base = jnp.dot(ltri16, tots.astype(jnp.bfloat16),
                   preferred_element_type=jnp.float32)
    cum = jnp.concatenate(
        [incs[ci] + base[ci:ci + 1, :] for ci in range(nch)], axis=0)

    counts = jnp.sum(tots, axis=0, keepdims=True)
    nblk = jnp.floor((counts + (BLK - 1)) / BLK)
    rc = lax.broadcasted_iota(jnp.int32, (NUM_EXPERTS, NUM_EXPERTS), 0)
    cc = lax.broadcasted_iota(jnp.int32, (NUM_EXPERTS, NUM_EXPERTS), 1)
    utri = jnp.where(rc < cc, 1.0, 0.0).astype(jnp.bfloat16)
    bstart = jnp.dot(nblk.astype(jnp.bfloat16), utri,
                     preferred_element_type=jnp.float32)
    po = bstart * BLK

    dest = po + cum
    pos0_ref[...] = jnp.sum(jnp.where(h1, dest, 0.0), axis=1,
                            keepdims=True).astype(jnp.int32)
    pos1_ref[...] = jnp.sum(jnp.where(h2, dest, 0.0), axis=1,
                            keepdims=True).astype(jnp.int32)

    nb_used = jnp.sum(nblk, axis=1, keepdims=True)
    bi = lax.broadcasted_iota(jnp.int32, (32, NUM_EXPERTS), 0).astype(
        jnp.float32)
    be = jnp.sum(jnp.where(bi >= bstart, 1.0, 0.0), axis=1,
                 keepdims=True) - 1.0
    bcol = lax.broadcasted_iota(jnp.int32, (32, 1), 0)
    meta_ref[...] = jnp.where(bcol < NBLK, be, nb_used).astype(jnp.int32)


def _gmm_body(meta_ref, xs_ref, w_ref, b_ref, y_ref):
    b = pl.program_id(0)

    @pl.when(b < meta_ref[NBLK])
    def _():
        e = meta_ref[b]
        y_ref[...] = jnp.dot(xs_ref[...], w_ref[e],
                             precision=lax.Precision.DEFAULT,
                             preferred_element_type=jnp.float32) + b_ref[e]


def _fma_body(g0_ref, g1_ref, w0_ref, w1_ref, out_ref):
    out_ref[...] = g0_ref[...] * w0_ref[...] + g1_ref[...] * w1_ref[...]


@jax.jit
def kernel(X, W_router, W_experts, b_experts):
    B, T, D = X.shape
    x2 = X.reshape(T, D)

    pos0, pos1, w0, w1, meta = pl.pallas_call(
        _router_meta_body,
        in_specs=[
            pl.BlockSpec((T, D), lambda: (0, 0)),
            pl.BlockSpec((D, NUM_EXPERTS), lambda: (0, 0)),
        ],
        out_specs=[
            pl.BlockSpec((T, 1), lambda: (0, 0)),
            pl.BlockSpec((T, 1), lambda: (0, 0)),
            pl.BlockSpec((T, 1), lambda: (0, 0)),
            pl.BlockSpec((T, 1), lambda: (0, 0)),
            pl.BlockSpec((32, 1), lambda: (0, 0)),
        ],
        out_shape=[
            jax.ShapeDtypeStruct((T, 1), jnp.int32),
            jax.ShapeDtypeStruct((T, 1), jnp.int32),
            jax.ShapeDtypeStruct((T, 1), jnp.float32),
            jax.ShapeDtypeStruct((T, 1), jnp.float32),
            jax.ShapeDtypeStruct((32, 1), jnp.int32),
        ],
    )(x2, W_router)

    info = plsc.get_sparse_core_info()
    nc, ns = info.num_cores, info.num_subcores
    nw = nc * ns
    tpw = T // nw
    mesh = plsc.VectorSubcoreMesh(core_axis_name="c", subcore_axis_name="s")
    pos0w = pos0.reshape(nw, tpw)
    pos1w = pos1.reshape(nw, tpw)

    @functools.partial(
        pl.kernel, mesh=mesh,
        out_type=jax.ShapeDtypeStruct((NPAD, D), jnp.float32),
        scratch_types=[
            pltpu.VMEM((tpw,), jnp.int32),
            pltpu.VMEM((tpw,), jnp.int32),
            pltpu.VMEM((tpw, D), jnp.float32),
            pltpu.SemaphoreType.DMA,
            pltpu.SemaphoreType.DMA,
        ],
    )
    def dispatch(x_hbm, p0_hbm, p1_hbm, xs_hbm, idx0_v, idx1_v, rows_v,
                 sem0, sem1):
        wid = lax.axis_index("s") * nc + lax.axis_index("c")
        base = wid * tpw
        i0 = pltpu.async_copy(p0_hbm.at[wid], idx0_v, sem0)
        i1 = pltpu.async_copy(p1_hbm.at[wid], idx1_v, sem1)
        pltpu.sync_copy(x_hbm.at[pl.ds(base, tpw)], rows_v)
        i0.wait()
        i1.wait()
        s0 = pltpu.async_copy(rows_v, xs_hbm.at[idx0_v], sem0)
        s1 = pltpu.async_copy(rows_v, xs_hbm.at[idx1_v], sem1)
        s0.wait()
        s1.wait()

    xs = dispatch(x2, pos0w, pos1w)

    y = pl.pallas_call(
        _gmm_body,
        grid_spec=pltpu.PrefetchScalarGridSpec(
            num_scalar_prefetch=1,
            grid=(NBLK,),
            in_specs=[
                pl.BlockSpec((BLK, D), lambda b, m: (b, 0)),
                pl.BlockSpec((NUM_EXPERTS, D, D), lambda b, m: (0, 0, 0)),
                pl.BlockSpec((NUM_EXPERTS, 1, D), lambda b, m: (0, 0, 0)),
            ],
            out_specs=pl.BlockSpec((BLK, D), lambda b, m: (b, 0)),
        ),
        out_shape=jax.ShapeDtypeStruct((NPAD, D), jnp.float32),
    )(meta.reshape(32), xs, W_experts, b_experts.reshape(NUM_EXPERTS, 1, D))

    @functools.partial(
        pl.kernel, mesh=mesh,
        out_type=(jax.ShapeDtypeStruct((T, D), jnp.float32),
                  jax.ShapeDtypeStruct((T, D), jnp.float32)),
        scratch_types=[
            pltpu.VMEM((tpw,), jnp.int32),
            pltpu.VMEM((tpw, D), jnp.float32),
            pltpu.SemaphoreType.DMA,
        ],
    )
    def combine_gather(y_hbm, p0_hbm, p1_hbm, g0_hbm, g1_hbm,
                       idx_v, rows_v, sem):
        wid = lax.axis_index("s") * nc + lax.axis_index("c")
        base = wid * tpw
        pltpu.sync_copy(p0_hbm.at[wid], idx_v)
        pltpu.async_copy(y_hbm.at[idx_v], rows_v, sem).wait()
        pltpu.sync_copy(rows_v, g0_hbm.at[pl.ds(base, tpw)])
        pltpu.sync_copy(p1_hbm.at[wid], idx_v)
        pltpu.async_copy(y_hbm.at[idx_v], rows_v, sem).wait()
        pltpu.sync_copy(rows_v, g1_hbm.at[pl.ds(base, tpw)])

    g0, g1 = combine_gather(y, pos0w, pos1w)

    out = pl.pallas_call(
        _fma_body,
        grid=(8,),
        in_specs=[
            pl.BlockSpec((T // 8, D), lambda i: (i, 0)),
            pl.BlockSpec((T // 8, D), lambda i: (i, 0)),
            pl.BlockSpec((T // 8, 1), lambda i: (i, 0)),
            pl.BlockSpec((T // 8, 1), lambda i: (i, 0)),
        ],
        out_specs=pl.BlockSpec((T // 8, D), lambda i: (i, 0)),
        out_shape=jax.ShapeDtypeStruct((T, D), jnp.float32),
    )(g0, g1, w0, w1)

    return out.reshape(B, T, D)


# speedup vs baseline: 5.6776x; 1.1479x over previous
import functools

import jax
import jax.numpy as jnp
from jax import lax
from jax.experimental import pallas as pl
from jax.experimental.pallas import tpu as pltpu
from jax.experimental.pallas import tpu_sc as plsc

NUM_EXPERTS = 8
TOP_K = 2
BLK = 256
NBLK = 24
NPAD = NBLK * BLK
CHUNK = 128


def _pack_pair(a_f32, b_f32):
    ai = lax.bitcast_convert_type(a_f32, jnp.int32) + jnp.int32(0x8000)
    bi = lax.bitcast_convert_type(b_f32, jnp.int32) + jnp.int32(0x8000)
    packed = jnp.bitwise_or(jnp.bitwise_and(bi, jnp.int32(-65536)),
                            lax.shift_right_logical(ai, 16))
    return lax.bitcast_convert_type(packed, jnp.float32)


def _unpack_pair(p_f32):
    p = lax.bitcast_convert_type(p_f32, jnp.int32)
    a = lax.bitcast_convert_type(jnp.left_shift(p, 16), jnp.float32)
    b = lax.bitcast_convert_type(jnp.bitwise_and(p, jnp.int32(-65536)),
                                 jnp.float32)
    return a, b


def _router_meta_body(x_ref, wr_ref, pos0_ref, pos1_ref, w0_ref, w1_ref,
                      meta_ref, xp_ref):
    x = x_ref[...]
    h = x.shape[1] // 2
    xp_ref[...] = _pack_pair(x[:, :h], x[:, h:])
    logits = jnp.dot(x, wr_ref[...], preferred_element_type=jnp.float32)
    t, ne = logits.shape
    eids = lax.broadcasted_iota(jnp.int32, (t, ne), 1)
    m1 = jnp.max(logits, axis=1, keepdims=True)
    a1 = jnp.min(jnp.where(logits == m1, eids, ne), axis=1, keepdims=True)
    h1 = eids == a1
    masked = jnp.where(h1, -jnp.inf, logits)
    m2 = jnp.max(masked, axis=1, keepdims=True)
    a2 = jnp.min(jnp.where(masked == m2, eids, ne), axis=1, keepdims=True)
    h2 = eids == a2
    e2 = jnp.exp(m2 - m1)
    w0_ref[...] = 1.0 / (1.0 + e2)
    w1_ref[...] = e2 / (1.0 + e2)

    cnt = jnp.where(h1, 1.0, 0.0) + jnp.where(h2, 1.0, 0.0)
    r = lax.broadcasted_iota(jnp.int32, (CHUNK, CHUNK), 0)
    c = lax.broadcasted_iota(jnp.int32, (CHUNK, CHUNK), 1)
    ltri = jnp.where(r > c, 1.0, 0.0).astype(jnp.bfloat16)
    nch = t // CHUNK
    incs = []
    tots = []
    for ci in range(nch):
        sl = cnt[ci * CHUNK:(ci + 1) * CHUNK, :]
        incs.append(jnp.dot(ltri, sl.astype(jnp.bfloat16),
                            preferred_element_type=jnp.float32))
        tots.append(jnp.sum(sl, axis=0, keepdims=True))
    tots = jnp.concatenate(tots, axis=0)
    r16 = lax.broadcasted_iota(jnp.int32, (nch, nch), 0)
    c16 = lax.broadcasted_iota(jnp.int32, (nch, nch), 1)
    ltri16 = jnp.where(r16 > c16, 1.0, 0.0).astype(jnp.bfloat16)
    base = jnp.dot(ltri16, tots.astype(jnp.bfloat16),
                   preferred_element_type=jnp.float32)
    cum = jnp.concatenate(
        [incs[ci] + base[ci:ci + 1, :] for ci in range(nch)], axis=0)

    counts = jnp.sum(tots, axis=0, keepdims=True)
    nblk = jnp.maximum(jnp.floor((counts + (BLK - 1)) / BLK), 1.0)
    rc = lax.broadcasted_iota(jnp.int32, (NUM_EXPERTS, NUM_EXPERTS), 0)
    cc = lax.broadcasted_iota(jnp.int32, (NUM_EXPERTS, NUM_EXPERTS), 1)
    utri = jnp.where(rc < cc, 1.0, 0.0).astype(jnp.bfloat16)
    bstart = jnp.dot(nblk.astype(jnp.bfloat16), utri,
                     preferred_element_type=jnp.float32)
    po = bstart * BLK

    dest = po + cum
    pos0_ref[...] = jnp.sum(jnp.where(h1, dest, 0.0), axis=1,
                            keepdims=True).astype(jnp.int32)
    pos1_ref[...] = jnp.sum(jnp.where(h2, dest, 0.0), axis=1,
                            keepdims=True).astype(jnp.int32)

    nb_used = jnp.sum(nblk, axis=1, keepdims=True)
    bi = lax.broadcasted_iota(jnp.int32, (32, NUM_EXPERTS), 0).astype(
        jnp.float32)
    be = jnp.sum(jnp.where(bi >= bstart, 1.0, 0.0), axis=1,
                 keepdims=True) - 1.0
    bcol = lax.broadcasted_iota(jnp.int32, (32, 1), 0)
    meta_ref[...] = jnp.where(bcol < NBLK, be, nb_used).astype(jnp.int32)


def _gmm_body(meta_ref, xs_ref, w_hbm, b_ref, y_ref, w_vmem, sems):
    b = pl.program_id(0)

    @pl.when(b == 0)
    def _start_dmas():
        for e in range(NUM_EXPERTS):
            pltpu.make_async_copy(w_hbm.at[e], w_vmem.at[e],
                                  sems.at[e]).start()

    e = meta_ref[b]
    prev = meta_ref[jnp.maximum(b - 1, 0)]
    for ec in range(NUM_EXPERTS):
        @pl.when(jnp.logical_and(e == ec,
                                 jnp.logical_or(b == 0, prev != ec)))
        def _wait_dma(ec=ec):
            pltpu.make_async_copy(w_hbm.at[ec], w_vmem.at[ec],
                                  sems.at[ec]).wait()

    @pl.when(b < meta_ref[NBLK])
    def _():
        a, bb = _unpack_pair(xs_ref[...])
        lhs = jnp.concatenate([a, bb], axis=1)
        y = jnp.dot(lhs, w_vmem[e],
                    precision=lax.Precision.DEFAULT,
                    preferred_element_type=jnp.float32) + b_ref[e]
        h = y.shape[1] // 2
        y_ref[...] = _pack_pair(y[:, :h], y[:, h:])


def _fma_body(g0_ref, g1_ref, w0_ref, w1_ref, out_ref):
    a0, b0 = _unpack_pair(g0_ref[...])
    a1, b1 = _unpack_pair(g1_ref[...])
    w0 = w0_ref[...]
    w1 = w1_ref[...]
    h = a0.shape[1]
    out_ref[:, :h] = a0 * w0 + a1 * w1
    out_ref[:, h:] = b0 * w0 + b1 * w1


@jax.jit
def kernel(X, W_router, W_experts, b_experts):
    B, T, D = X.shape
    x2 = X.reshape(T, D)

    d2 = D // 2
    pos0, pos1, w0, w1, meta, xp = pl.pallas_call(
        _router_meta_body,
        in_specs=[
            pl.BlockSpec((T, D), lambda: (0, 0)),
            pl.BlockSpec((D, NUM_EXPERTS), lambda: (0, 0)),
        ],
        out_specs=[
            pl.BlockSpec((T, 1), lambda: (0, 0)),
            pl.BlockSpec((T, 1), lambda: (0, 0)),
            pl.BlockSpec((T, 1), lambda: (0, 0)),
            pl.BlockSpec((T, 1), lambda: (0, 0)),
            pl.BlockSpec((32, 1), lambda: (0, 0)),
            pl.BlockSpec((T, d2), lambda: (0, 0)),
        ],
        out_shape=[
            jax.ShapeDtypeStruct((T, 1), jnp.int32),
            jax.ShapeDtypeStruct((T, 1), jnp.int32),
            jax.ShapeDtypeStruct((T, 1), jnp.float32),
            jax.ShapeDtypeStruct((T, 1), jnp.float32),
            jax.ShapeDtypeStruct((32, 1), jnp.int32),
            jax.ShapeDtypeStruct((T, d2), jnp.float32),
        ],
    )(x2, W_router)

    info = plsc.get_sparse_core_info()
    nc, ns = info.num_cores, info.num_subcores
    nw = nc * ns
    tpw = T // nw
    mesh = plsc.VectorSubcoreMesh(core_axis_name="c", subcore_axis_name="s")
    pos0w = pos0.reshape(nw, tpw)
    pos1w = pos1.reshape(nw, tpw)

    @functools.partial(
        pl.kernel, mesh=mesh,
        out_type=jax.ShapeDtypeStruct((NPAD, d2), jnp.float32),
        scratch_types=[
            pltpu.VMEM((tpw,), jnp.int32),
            pltpu.VMEM((tpw,), jnp.int32),
            pltpu.VMEM((tpw, d2), jnp.float32),
            pltpu.SemaphoreType.DMA,
            pltpu.SemaphoreType.DMA,
        ],
    )
    def dispatch(x_hbm, p0_hbm, p1_hbm, xs_hbm, idx0_v, idx1_v, rows_v,
                 sem0, sem1):
        wid = lax.axis_index("s") * nc + lax.axis_index("c")
        base = wid * tpw
        i0 = pltpu.async_copy(p0_hbm.at[wid], idx0_v, sem0)
        i1 = pltpu.async_copy(p1_hbm.at[wid], idx1_v, sem1)
        pltpu.sync_copy(x_hbm.at[pl.ds(base, tpw)], rows_v)
        i0.wait()
        i1.wait()
        s0 = pltpu.async_copy(rows_v, xs_hbm.at[idx0_v], sem0)
        s1 = pltpu.async_copy(rows_v, xs_hbm.at[idx1_v], sem1)
        s0.wait()
        s1.wait()

    xs = dispatch(xp, pos0w, pos1w)

    y = pl.pallas_call(
        _gmm_body,
        grid_spec=pltpu.PrefetchScalarGridSpec(
            num_scalar_prefetch=1,
            grid=(NBLK,),
            in_specs=[
                pl.BlockSpec((BLK, d2), lambda b, m: (b, 0)),
                pl.BlockSpec(memory_space=pl.ANY),
                pl.BlockSpec((NUM_EXPERTS, 1, D), lambda b, m: (0, 0, 0)),
            ],
            out_specs=pl.BlockSpec((BLK, d2), lambda b, m: (b, 0)),
            scratch_shapes=[
                pltpu.VMEM((NUM_EXPERTS, D, D), jnp.float32),
                pltpu.SemaphoreType.DMA((NUM_EXPERTS,)),
            ],
        ),
        out_shape=jax.ShapeDtypeStruct((NPAD, d2), jnp.float32),
    )(meta.reshape(32), xs, W_experts, b_experts.reshape(NUM_EXPERTS, 1, D))

    @functools.partial(
        pl.kernel, mesh=mesh,
        out_type=(jax.ShapeDtypeStruct((T, d2), jnp.float32),
                  jax.ShapeDtypeStruct((T, d2), jnp.float32)),
        scratch_types=[
            pltpu.VMEM((tpw,), jnp.int32),
            pltpu.VMEM((tpw, d2), jnp.float32),
            pltpu.SemaphoreType.DMA,
        ],
    )
    def combine_gather(y_hbm, p0_hbm, p1_hbm, g0_hbm, g1_hbm,
                       idx_v, rows_v, sem):
        wid = lax.axis_index("s") * nc + lax.axis_index("c")
        base = wid * tpw
        pltpu.sync_copy(p0_hbm.at[wid], idx_v)
        pltpu.async_copy(y_hbm.at[idx_v], rows_v, sem).wait()
        pltpu.sync_copy(rows_v, g0_hbm.at[pl.ds(base, tpw)])
        pltpu.sync_copy(p1_hbm.at[wid], idx_v)
        pltpu.async_copy(y_hbm.at[idx_v], rows_v, sem).wait()
        pltpu.sync_copy(rows_v, g1_hbm.at[pl.ds(base, tpw)])

    g0, g1 = combine_gather(y, pos0w, pos1w)

    out = pl.pallas_call(
        _fma_body,
        grid=(8,),
        in_specs=[
            pl.BlockSpec((T // 8, d2), lambda i: (i, 0)),
            pl.BlockSpec((T // 8, d2), lambda i: (i, 0)),
            pl.BlockSpec((T // 8, 1), lambda i: (i, 0)),
            pl.BlockSpec((T // 8, 1), lambda i: (i, 0)),
        ],
        out_specs=pl.BlockSpec((T // 8, D), lambda i: (i, 0)),
        out_shape=jax.ShapeDtypeStruct((T, D), jnp.float32),
    )(g0, g1, w0, w1)

    return out.reshape(B, T, D)
